# baseline (speedup 1.0000x reference)
import functools

import jax
import jax.numpy as jnp
from jax import lax
from jax.experimental import pallas as pl
from jax.experimental.pallas import tpu as pltpu
from jax.experimental.pallas import tpu_sc as plsc

BT = 4096
D = 1024
E = 8
RBT = 2048
NBR = BT // RBT
BMK = 128
R0 = BT * 2 + E * BMK
NT = R0 // BMK
NW = 32
TPW = BT // NW


def _route_body(x_ref, wg_ref, posm_ref, wm_ref, gid_ref, meta_scr, cntb_scr):
    p = pl.program_id(0)
    i = pl.program_id(1)

    @pl.when(p == 0)
    def _():
        xb = x_ref[...]
        logits = jax.lax.dot_general(
            xb, wg_ref[...], (((1,), (1,)), ((), ())),
            preferred_element_type=jnp.float32)
        m = jnp.max(logits, axis=1, keepdims=True)
        s = jnp.exp(logits - m)
        gate = s / jnp.sum(s, axis=1, keepdims=True)
        iota = lax.broadcasted_iota(jnp.int32, gate.shape, 1)
        v1 = jnp.max(gate, axis=1, keepdims=True)
        i1 = jnp.min(jnp.where(gate == v1, iota, E), axis=1, keepdims=True)
        g2 = jnp.where(iota == i1, -jnp.inf, gate)
        v2 = jnp.max(g2, axis=1, keepdims=True)
        i2 = jnp.min(jnp.where(g2 == v2, iota, E), axis=1, keepdims=True)
        wsum = v1 + v2 + 1e-9
        w1 = v1 / wsum
        w2 = v2 / wsum
        oh = (jnp.where(iota == i1, 1.0, 0.0)
              + jnp.where(iota == i2, 1.0, 0.0))
        cs = oh
        sft = 1
        while sft < RBT:
            z = jnp.zeros((sft, E), jnp.float32)
            cs = cs + jnp.concatenate([z, cs[:RBT - sft]], axis=0)
            sft *= 2
        ex = cs - oh
        r1 = jnp.sum(jnp.where(iota == i1, ex, 0.0), axis=1, keepdims=True)
        r2 = jnp.sum(jnp.where(iota == i2, ex, 0.0), axis=1, keepdims=True)
        meta = jnp.concatenate(
            [w1, w2, i1.astype(jnp.float32), i2.astype(jnp.float32),
             r1, r2, jnp.zeros((RBT, 2), jnp.float32)], axis=1)
        meta_scr[pl.ds(i * RBT, RBT), :] = meta
        csum = jnp.sum(oh, axis=0, keepdims=True)
        cntb_scr[pl.ds(i * 8, 1), :] = csum

    @pl.when(p == 1)
    def _():
        meta = meta_scr[pl.ds(i * RBT, RBT), :]
        w1 = meta[:, 0:1]
        w2 = meta[:, 1:2]
        i1 = meta[:, 2:3].astype(jnp.int32)
        i2 = meta[:, 3:4].astype(jnp.int32)
        r1 = meta[:, 4:5]
        r2 = meta[:, 5:6]
        cnt0 = cntb_scr[0:1, :]
        cnt1 = cntb_scr[8:9, :]
        total = cnt0 + cnt1
        carry = cnt0 * (i > 0).astype(jnp.float32)
        pc = jnp.floor((total + (BMK - 1)) * (1.0 / BMK)) * BMK
        offs = [jnp.zeros((1, 1), jnp.float32)]
        run = jnp.zeros((1, 1), jnp.float32)
        for e in range(1, E):
            run = run + pc[:, e - 1:e]
            offs.append(run)
        off = jnp.concatenate(offs, axis=1)
        lane = (lax.broadcasted_iota(jnp.int32, (1, 128), 1)
                .astype(jnp.float32) * float(BMK))
        acc = jnp.zeros((1, 128), jnp.float32)
        for e in range(E):
            acc = acc + jnp.where(lane >= off[:, e:e + 1], 1.0, 0.0)
        gid_ref[...] = (acc - 1.0).astype(jnp.int32)
        iota = lax.broadcasted_iota(jnp.int32, (RBT, E), 1)
        base = off + carry
        b1 = jnp.sum(jnp.where(iota == i1, base, 0.0), axis=1, keepdims=True)
        b2 = jnp.sum(jnp.where(iota == i2, base, 0.0), axis=1, keepdims=True)
        pos1 = (b1 + r1).astype(jnp.int32)
        pos2 = (b2 + r2).astype(jnp.int32)
        posm_ref[...] = jnp.concatenate(
            [pos1.reshape(1, RBT), pos2.reshape(1, RBT)], axis=0)
        w1_16 = jnp.concatenate([w1] * 16, axis=1).reshape(1, RBT, 16)
        w2_16 = jnp.concatenate([w2] * 16, axis=1).reshape(1, RBT, 16)
        wm_ref[...] = jnp.concatenate([w1_16, w2_16], axis=0)


def _route(xf, W_gate):
    return pl.pallas_call(
        _route_body,
        grid=(2, NBR),
        in_specs=[
            pl.BlockSpec((RBT, D), lambda p, i: (i * (1 - p), 0)),
            pl.BlockSpec((E, D), lambda p, i: (0, 0)),
        ],
        out_specs=[
            pl.BlockSpec((2, RBT), lambda p, i: (0, i)),
            pl.BlockSpec((2, RBT, 16), lambda p, i: (0, i, 0)),
            pl.BlockSpec((1, 128), lambda p, i: (0, 0)),
        ],
        out_shape=[
            jax.ShapeDtypeStruct((2, BT), jnp.int32),
            jax.ShapeDtypeStruct((2, BT, 16), jnp.float32),
            jax.ShapeDtypeStruct((1, 128), jnp.int32),
        ],
        scratch_shapes=[
            pltpu.VMEM((BT, 8), jnp.float32),
            pltpu.VMEM((16, 8), jnp.float32),
        ],
    )(xf, W_gate)


def _dispatch_body(x_hbm, posm_hbm, xs_hbm, idx0_v, idx1_v, rows_v, sem):
    wid = lax.axis_index("s") * 2 + lax.axis_index("c")
    base = wid * TPW
    for c in range(TPW // 64):
        off = base + c * 64
        pltpu.sync_copy(posm_hbm.at[0, pl.ds(off, 64)], idx0_v)
        pltpu.sync_copy(posm_hbm.at[1, pl.ds(off, 64)], idx1_v)
        pltpu.sync_copy(x_hbm.at[pl.ds(off, 64)], rows_v)
        pltpu.async_copy(rows_v, xs_hbm.at[idx0_v], sem).wait()
        pltpu.async_copy(rows_v, xs_hbm.at[idx1_v], sem).wait()


def _dispatch(xf, posm):
    mesh = plsc.VectorSubcoreMesh(core_axis_name="c", subcore_axis_name="s")
    f = functools.partial(
        pl.kernel,
        mesh=mesh,
        out_type=jax.ShapeDtypeStruct((R0, D), jnp.float32),
        scratch_types=[
            pltpu.VMEM((64,), jnp.int32),
            pltpu.VMEM((64,), jnp.int32),
            pltpu.VMEM((64, D), jnp.float32),
            pltpu.SemaphoreType.DMA,
        ],
    )(_dispatch_body)
    return f(xf, posm)


def _gmm_body(gid_ref, x_ref, we_ref, y_ref):
    y_ref[...] = jax.lax.dot_general(
        x_ref[...], we_ref[0], (((1,), (0,)), ((), ())),
        preferred_element_type=jnp.float32)


def _gmm(gid, xs, We):
    grid_spec = pltpu.PrefetchScalarGridSpec(
        num_scalar_prefetch=1,
        grid=(NT,),
        in_specs=[
            pl.BlockSpec((BMK, D), lambda i, gid: (i, 0)),
            pl.BlockSpec((1, D, D), lambda i, gid: (gid[i], 0, 0)),
        ],
        out_specs=pl.BlockSpec((BMK, D), lambda i, gid: (i, 0)),
    )
    return pl.pallas_call(
        _gmm_body,
        grid_spec=grid_spec,
        out_shape=jax.ShapeDtypeStruct((R0, D), jnp.float32),
    )(gid, xs, We)


CH = 16


def _combine_body(y_hbm, posm_hbm, wm_hbm, out_hbm,
                  idx0_v, idx1_v, w0_v, w1_v, y0_v, y1_v, out_v, sem):
    wid = lax.axis_index("s") * 2 + lax.axis_index("c")
    base = wid * TPW

    def chunk(c, _):
        off = base + c * CH
        pltpu.sync_copy(posm_hbm.at[0, pl.ds(off, CH)], idx0_v)
        pltpu.sync_copy(posm_hbm.at[1, pl.ds(off, CH)], idx1_v)
        pltpu.sync_copy(wm_hbm.at[0, pl.ds(off, CH)], w0_v)
        pltpu.sync_copy(wm_hbm.at[1, pl.ds(off, CH)], w1_v)
        pltpu.async_copy(y_hbm.at[idx0_v], y0_v, sem).wait()
        pltpu.async_copy(y_hbm.at[idx1_v], y1_v, sem).wait()
        for t in range(CH):
            wv0 = w0_v[t, :]
            wv1 = w1_v[t, :]
            for g in range(D // 16):
                a = y0_v[t, pl.ds(g * 16, 16)]
                b = y1_v[t, pl.ds(g * 16, 16)]
                out_v[t, pl.ds(g * 16, 16)] = a * wv0 + b * wv1
        pltpu.sync_copy(out_v, out_hbm.at[pl.ds(off, CH)])
        return 0

    lax.fori_loop(0, TPW // CH, chunk, 0)


def _combine(y, posm, wm):
    mesh = plsc.VectorSubcoreMesh(core_axis_name="c", subcore_axis_name="s")
    f = functools.partial(
        pl.kernel,
        mesh=mesh,
        out_type=jax.ShapeDtypeStruct((BT, D), jnp.float32),
        scratch_types=[
            pltpu.VMEM((CH,), jnp.int32),
            pltpu.VMEM((CH,), jnp.int32),
            pltpu.VMEM((CH, 16), jnp.float32),
            pltpu.VMEM((CH, 16), jnp.float32),
            pltpu.VMEM((CH, D), jnp.float32),
            pltpu.VMEM((CH, D), jnp.float32),
            pltpu.VMEM((CH, D), jnp.float32),
            pltpu.SemaphoreType.DMA,
        ],
    )(_combine_body)
    return f(y, posm, wm)


def kernel(x, W_gate, We):
    B, T, Dm = x.shape
    xf = x.reshape(B * T, Dm)
    posm, wm, gid = _route(xf, W_gate)
    xs = _dispatch(xf, posm)
    y = _gmm(gid.reshape(128), xs, We)
    out = _combine(y, posm, wm)
    return out.reshape(B, T, Dm)


# speedup vs baseline: 2.3566x; 2.3566x over previous
import jax
import jax.numpy as jnp
from jax.experimental import pallas as pl
from jax.experimental.pallas import tpu as pltpu


def _moe_dense_body(x_ref, wg_ref, we_ref, o_ref, w_scr):
    e = pl.program_id(1)
    nE = pl.num_programs(1)

    @pl.when(e == 0)
    def _():
        xb = x_ref[...]
        logits = jax.lax.dot_general(
            xb, wg_ref[...], (((1,), (1,)), ((), ())),
            preferred_element_type=jnp.float32)
        m = jnp.max(logits, axis=1, keepdims=True)
        s = jnp.exp(logits - m)
        gate = s / jnp.sum(s, axis=1, keepdims=True)
        iota = jax.lax.broadcasted_iota(jnp.int32, gate.shape, 1)
        v1 = jnp.max(gate, axis=1, keepdims=True)
        i1 = jnp.min(jnp.where(gate == v1, iota, nE), axis=1, keepdims=True)
        g2 = jnp.where(iota == i1, -jnp.inf, gate)
        v2 = jnp.max(g2, axis=1, keepdims=True)
        i2 = jnp.min(jnp.where(g2 == v2, iota, nE), axis=1, keepdims=True)
        wsum = v1 + v2 + 1e-9
        w = (jnp.where(iota == i1, v1 / wsum, 0.0)
             + jnp.where(iota == i2, v2 / wsum, 0.0))
        w_scr[...] = w

    contrib = jax.lax.dot_general(
        x_ref[...], we_ref[0], (((1,), (0,)), ((), ())),
        preferred_element_type=jnp.float32)
    wall = w_scr[...]
    eiota = jax.lax.broadcasted_iota(jnp.int32, wall.shape, 1)
    wcol = jnp.sum(jnp.where(eiota == e, wall, 0.0), axis=1, keepdims=True)
    contrib = contrib * wcol

    @pl.when(e == 0)
    def _():
        o_ref[...] = contrib

    @pl.when(e != 0)
    def _():
        o_ref[...] += contrib


def kernel(x, W_gate, We):
    B, T, D = x.shape
    E = We.shape[0]
    xf = x.reshape(B * T, D)
    BT_BLK = 2048
    grid = (B * T // BT_BLK, E)
    out = pl.pallas_call(
        _moe_dense_body,
        grid=grid,
        in_specs=[
            pl.BlockSpec((BT_BLK, D), lambda i, e: (i, 0)),
            pl.BlockSpec((E, D), lambda i, e: (0, 0)),
            pl.BlockSpec((1, D, D), lambda i, e: (e, 0, 0)),
        ],
        out_specs=pl.BlockSpec((BT_BLK, D), lambda i, e: (i, 0)),
        out_shape=jax.ShapeDtypeStruct((B * T, D), jnp.float32),
        scratch_shapes=[pltpu.VMEM((BT_BLK, E), jnp.float32)],
    )(xf, W_gate, We)
    return out.reshape(B, T, D)
